# baseline (speedup 1.0000x reference)
import functools

import jax
import jax.numpy as jnp
from jax import lax
from jax.experimental import pallas as pl
from jax.experimental.pallas import tpu as pltpu
from jax.experimental.pallas import tpu_sc as plsc

EMBED = 64
NC = 2
NS = 16
NW = NC * NS
CHUNK = 128


@functools.lru_cache(maxsize=None)
def _build(n, steps):
    mesh = plsc.VectorSubcoreMesh(core_axis_name="c", subcore_axis_name="s")

    @functools.partial(
        pl.kernel,
        out_type=jax.ShapeDtypeStruct((n, EMBED), jnp.float32),
        mesh=mesh,
        compiler_params=pltpu.CompilerParams(use_tc_tiling_on_sc=False),
        scratch_types=[
            pltpu.VMEM((steps, CHUNK), jnp.int32),
            pltpu.VMEM((CHUNK, EMBED), jnp.float32),
            pltpu.SemaphoreType.DMA,
        ],
    )
    def emb(x_hbm, table_hbm, out_hbm, idx_v, rows_v, gsem):
        wid = lax.axis_index("s") * NC + lax.axis_index("c")
        base = wid * steps * CHUNK
        pltpu.sync_copy(x_hbm.at[wid], idx_v)

        @pl.loop(0, steps)
        def _(i):
            pltpu.async_copy(table_hbm.at[idx_v.at[i]], rows_v, gsem).wait()
            pltpu.sync_copy(rows_v, out_hbm.at[pl.ds(base + i * CHUNK, CHUNK)])

    return emb


def kernel(x, table):
    B, T = x.shape
    n = B * T
    steps = n // (NW * CHUNK)
    xw = x.astype(jnp.int32).reshape(NW, steps, CHUNK)
    out = _build(n, steps)(xw, table)
    return out.reshape(B, T, EMBED)


# speedup vs baseline: 1.1151x; 1.1151x over previous
import functools

import jax
import jax.numpy as jnp
from jax import lax
from jax.experimental import pallas as pl
from jax.experimental.pallas import tpu as pltpu
from jax.experimental.pallas import tpu_sc as plsc

EMBED = 64
NC = 2
NS = 16
NW = NC * NS
CHUNK = 128
K = 4
BUF = K * CHUNK


@functools.lru_cache(maxsize=None)
def _build(n, steps):
    outer = steps // K
    per_w = steps * CHUNK
    mesh = plsc.VectorSubcoreMesh(core_axis_name="c", subcore_axis_name="s")

    @functools.partial(
        pl.kernel,
        out_type=jax.ShapeDtypeStruct((n, EMBED), jnp.float32),
        mesh=mesh,
        compiler_params=pltpu.CompilerParams(use_tc_tiling_on_sc=False),
        scratch_types=[
            pltpu.VMEM((steps, CHUNK), jnp.int32),
            pltpu.VMEM((BUF, EMBED), jnp.float32),
            pltpu.VMEM((BUF, EMBED), jnp.float32),
            pltpu.SemaphoreType.DMA,
            pltpu.SemaphoreType.DMA,
            pltpu.SemaphoreType.DMA,
        ],
    )
    def emb(x_hbm, table_hbm, out_hbm, idx_v, rows0, rows1, gsem0, gsem1, osem):
        wid = lax.axis_index("s") * NC + lax.axis_index("c")
        base = wid * per_w
        pltpu.sync_copy(x_hbm.at[wid], idx_v)

        bufs = (rows0, rows1)
        gsems = (gsem0, gsem1)

        def fire_gather(g, b):
            for j in range(K):
                pltpu.async_copy(
                    table_hbm.at[idx_v.at[g * K + j]],
                    bufs[b].at[pl.ds(j * CHUNK, CHUNK)],
                    gsems[b],
                )

        def drain_gather(b):
            pltpu.make_async_copy(
                out_hbm.at[pl.ds(0, BUF)], bufs[b], gsems[b]
            ).wait()

        fire_gather(0, 0)
        fire_gather(1, 1)

        @pl.loop(0, outer // 2)
        def _(p):
            for b in range(2):
                g = p * 2 + b
                drain_gather(b)
                pltpu.async_copy(
                    bufs[b], out_hbm.at[pl.ds(base + g * BUF, BUF)], osem
                ).wait()

                @pl.when(g + 2 < outer)
                def _():
                    fire_gather(g + 2, b)

    return emb


def kernel(x, table):
    B, T = x.shape
    n = B * T
    steps = n // (NW * CHUNK)
    xw = x.astype(jnp.int32).reshape(NW, steps, CHUNK)
    out = _build(n, steps)(xw, table)
    return out.reshape(B, T, EMBED)
